# Initial kernel scaffold; baseline (speedup 1.0000x reference)
import jax
import jax.numpy as jnp
from jax.experimental import pallas as pl


def kernel(x, enc_W, enc_b, codebook, dec_W, dec_b):
    raise NotImplementedError("write your pallas kernel here")


# speedup vs baseline: 1.0300x; 1.0300x over previous
import functools

import jax
import jax.numpy as jnp
from jax.experimental import pallas as pl
from jax.experimental.pallas import tpu as pltpu

_TILE_M = 2048
_HI = jax.lax.Precision.HIGHEST
_DEF = jax.lax.Precision.DEFAULT


def _vq_kernel(xv_ref, enc_w_ref, enc_b_ref, cb_ref, dec_w_ref, dec_b_ref,
               out_ref, cbsq_ref, cbdec_ref):
    @pl.when(pl.program_id(0) == 0)
    def _init():
        cb = cb_ref[...]
        cbsq_ref[...] = jnp.sum(cb * cb, axis=1, keepdims=True).reshape(1, -1)
        cbdec_ref[...] = jax.lax.dot_general(
            cb, dec_w_ref[...], (((1,), (1,)), ((), ())),
            preferred_element_type=jnp.float32, precision=_DEF)

    xv = xv_ref[...]
    z = jax.lax.dot_general(
        xv, enc_w_ref[...], (((1,), (1,)), ((), ())),
        preferred_element_type=jnp.float32, precision=_DEF)
    z = z + enc_b_ref[...]
    zc = jax.lax.dot_general(
        z, cb_ref[...], (((1,), (1,)), ((), ())),
        preferred_element_type=jnp.float32, precision=_DEF)
    zsq = jnp.sum(z * z, axis=1, keepdims=True)
    dists = zsq - 2.0 * zc + cbsq_ref[...]
    idx = jnp.argmin(dists, axis=1)
    onehot = (jax.lax.broadcasted_iota(jnp.int32, dists.shape, 1)
              == idx[:, None]).astype(jnp.float32)
    out = jax.lax.dot_general(
        onehot, cbdec_ref[...], (((1,), (0,)), ((), ())),
        preferred_element_type=jnp.float32, precision=_HI)
    out_ref[...] = out + dec_b_ref[...]


@functools.partial(jax.jit, static_argnames=())
def kernel(x, enc_W, enc_b, codebook, dec_W, dec_b):
    b, c, d_, h, w = x.shape
    k, emb = codebook.shape
    n = b * d_ * h * w
    xv = x.transpose(0, 2, 3, 4, 1).reshape(n, c)
    grid = (n // _TILE_M,)
    out = pl.pallas_call(
        _vq_kernel,
        grid=grid,
        in_specs=[
            pl.BlockSpec((_TILE_M, c), lambda i: (i, 0)),
            pl.BlockSpec((emb, c), lambda i: (0, 0)),
            pl.BlockSpec((1, emb), lambda i: (0, 0)),
            pl.BlockSpec((k, emb), lambda i: (0, 0)),
            pl.BlockSpec((c, emb), lambda i: (0, 0)),
            pl.BlockSpec((1, c), lambda i: (0, 0)),
        ],
        out_specs=pl.BlockSpec((_TILE_M, c), lambda i: (i, 0)),
        out_shape=jax.ShapeDtypeStruct((n, c), jnp.float32),
        scratch_shapes=[
            pltpu.VMEM((1, k), jnp.float32),
            pltpu.VMEM((k, c), jnp.float32),
        ],
    )(xv, enc_W, enc_b.reshape(1, emb), codebook, dec_W, dec_b.reshape(1, c))
    return out.reshape(b, d_, h, w, c).transpose(0, 4, 1, 2, 3)


# speedup vs baseline: 2.6133x; 2.5372x over previous
import functools

import jax
import jax.numpy as jnp
from jax.experimental import pallas as pl
from jax.experimental.pallas import tpu as pltpu

_TILE_M = 2048
_DEF = jax.lax.Precision.DEFAULT


def _vq_kernel(xv_ref, enc_w_ref, enc_b_ref, cb_ref, dec_w_ref, dec_b_ref,
               out_ref, cbm2_ref, cbsq_ref, cbdec_ref):
    @pl.when(pl.program_id(0) == 0)
    def _init():
        cb = cb_ref[...]
        cbm2_ref[...] = cb * -2.0
        cbsq_ref[...] = jnp.sum(cb * cb, axis=1, keepdims=True)
        cbdec_ref[...] = jax.lax.dot_general(
            cb, dec_w_ref[...], (((1,), (1,)), ((), ())),
            preferred_element_type=jnp.float32, precision=_DEF)

    xv = xv_ref[...]
    zt = jax.lax.dot_general(
        enc_w_ref[...], xv, (((1,), (1,)), ((), ())),
        preferred_element_type=jnp.float32, precision=_DEF)
    zt = zt + enc_b_ref[...]
    zcm2 = jax.lax.dot_general(
        cbm2_ref[...], zt, (((1,), (0,)), ((), ())),
        preferred_element_type=jnp.float32, precision=_DEF)
    zsq = jnp.sum(zt * zt, axis=0, keepdims=True)
    dists = (zsq + zcm2) + cbsq_ref[...]
    idx = jnp.argmin(dists, axis=0)
    onehot = (jax.lax.broadcasted_iota(jnp.int32, dists.shape, 0)
              == idx[None, :]).astype(jnp.float32)
    out = jax.lax.dot_general(
        onehot, cbdec_ref[...], (((0,), (0,)), ((), ())),
        preferred_element_type=jnp.float32, precision=_DEF)
    out_ref[...] = out + dec_b_ref[...]


@functools.partial(jax.jit, static_argnames=())
def kernel(x, enc_W, enc_b, codebook, dec_W, dec_b):
    b, c, d_, h, w = x.shape
    k, emb = codebook.shape
    n = b * d_ * h * w
    xv = x.transpose(0, 2, 3, 4, 1).reshape(n, c)
    grid = (n // _TILE_M,)
    out = pl.pallas_call(
        _vq_kernel,
        grid=grid,
        in_specs=[
            pl.BlockSpec((_TILE_M, c), lambda i: (i, 0)),
            pl.BlockSpec((emb, c), lambda i: (0, 0)),
            pl.BlockSpec((emb, 1), lambda i: (0, 0)),
            pl.BlockSpec((k, emb), lambda i: (0, 0)),
            pl.BlockSpec((c, emb), lambda i: (0, 0)),
            pl.BlockSpec((1, c), lambda i: (0, 0)),
        ],
        out_specs=pl.BlockSpec((_TILE_M, c), lambda i: (i, 0)),
        out_shape=jax.ShapeDtypeStruct((n, c), jnp.float32),
        scratch_shapes=[
            pltpu.VMEM((k, emb), jnp.float32),
            pltpu.VMEM((k, 1), jnp.float32),
            pltpu.VMEM((k, c), jnp.float32),
        ],
    )(xv, enc_W, enc_b.reshape(emb, 1), codebook, dec_W, dec_b.reshape(1, c))
    return out.reshape(b, d_, h, w, c).transpose(0, 4, 1, 2, 3)


# speedup vs baseline: 4.1838x; 1.6010x over previous
import functools

import jax
import jax.numpy as jnp
from jax.experimental import pallas as pl
from jax.experimental.pallas import tpu as pltpu

_TILE_S = 2048
_DEF = jax.lax.Precision.DEFAULT


def _vq_kernel(xv_ref, enc_w_ref, enc_b_ref, cb_ref, dec_w_ref, dec_b_ref,
               out_ref, cbm2_ref, cbsq_ref, cbdec_ref):
    @pl.when((pl.program_id(0) == 0) & (pl.program_id(1) == 0))
    def _init():
        cb = cb_ref[...]
        cbm2_ref[...] = cb * -2.0
        cbsq_ref[...] = jnp.sum(cb * cb, axis=1, keepdims=True)
        cbdec_ref[...] = jax.lax.dot_general(
            cb, dec_w_ref[...], (((1,), (1,)), ((), ())),
            preferred_element_type=jnp.float32, precision=_DEF)

    xv = xv_ref[0]
    zt = jax.lax.dot_general(
        enc_w_ref[...], xv, (((1,), (0,)), ((), ())),
        preferred_element_type=jnp.float32, precision=_DEF)
    zt = zt + enc_b_ref[...]
    zcm2 = jax.lax.dot_general(
        cbm2_ref[...], zt, (((1,), (0,)), ((), ())),
        preferred_element_type=jnp.float32, precision=_DEF)
    zsq = jnp.sum(zt * zt, axis=0, keepdims=True)
    dists = (zsq + zcm2) + cbsq_ref[...]
    idx = jnp.argmin(dists, axis=0)
    onehot = (jax.lax.broadcasted_iota(jnp.int32, dists.shape, 0)
              == idx[None, :]).astype(jnp.float32)
    out = jax.lax.dot_general(
        cbdec_ref[...], onehot, (((0,), (0,)), ((), ())),
        preferred_element_type=jnp.float32, precision=_DEF)
    out_ref[0] = out + dec_b_ref[...]


@functools.partial(jax.jit, static_argnames=())
def kernel(x, enc_W, enc_b, codebook, dec_W, dec_b):
    b, c, d_, h, w = x.shape
    k, emb = codebook.shape
    s = d_ * h * w
    xv = x.reshape(b, c, s)
    grid = (b, s // _TILE_S)
    out = pl.pallas_call(
        _vq_kernel,
        grid=grid,
        in_specs=[
            pl.BlockSpec((1, c, _TILE_S), lambda bi, si: (bi, 0, si)),
            pl.BlockSpec((emb, c), lambda bi, si: (0, 0)),
            pl.BlockSpec((emb, 1), lambda bi, si: (0, 0)),
            pl.BlockSpec((k, emb), lambda bi, si: (0, 0)),
            pl.BlockSpec((c, emb), lambda bi, si: (0, 0)),
            pl.BlockSpec((c, 1), lambda bi, si: (0, 0)),
        ],
        out_specs=pl.BlockSpec((1, c, _TILE_S), lambda bi, si: (bi, 0, si)),
        out_shape=jax.ShapeDtypeStruct((b, c, s), jnp.float32),
        scratch_shapes=[
            pltpu.VMEM((k, emb), jnp.float32),
            pltpu.VMEM((k, 1), jnp.float32),
            pltpu.VMEM((k, c), jnp.float32),
        ],
    )(xv, enc_W, enc_b.reshape(emb, 1), codebook, dec_W, dec_b.reshape(c, 1))
    return out.reshape(b, c, d_, h, w)


# speedup vs baseline: 4.1865x; 1.0006x over previous
import functools

import jax
import jax.numpy as jnp
from jax.experimental import pallas as pl
from jax.experimental.pallas import tpu as pltpu

_TILE_S = 2048
_DEF = jax.lax.Precision.DEFAULT


def _vq_kernel(xv_ref, enc_w_ref, enc_b_ref, cb_ref, dec_w_ref, dec_b_ref,
               out_ref, cbm2_ref, cbsq_ref, cbdec_ref):
    @pl.when((pl.program_id(0) == 0) & (pl.program_id(1) == 0))
    def _init():
        cb = cb_ref[...]
        cbm2_ref[...] = cb * -2.0
        cbsq_ref[...] = jnp.sum(cb * cb, axis=1, keepdims=True)
        cbdec_ref[...] = jax.lax.dot_general(
            cb, dec_w_ref[...], (((1,), (1,)), ((), ())),
            preferred_element_type=jnp.float32, precision=_DEF)

    xv = xv_ref[0]
    zt = jax.lax.dot_general(
        enc_w_ref[...], xv, (((1,), (0,)), ((), ())),
        preferred_element_type=jnp.float32, precision=_DEF)
    zt = zt + enc_b_ref[...]
    zcm2 = jax.lax.dot_general(
        cbm2_ref[...], zt, (((1,), (0,)), ((), ())),
        preferred_element_type=jnp.float32, precision=_DEF)
    zsq = jnp.sum(zt * zt, axis=0, keepdims=True)
    dists = (zsq + zcm2) + cbsq_ref[...]
    idx = jnp.argmin(dists, axis=0)
    onehot = (jax.lax.broadcasted_iota(jnp.int32, dists.shape, 0)
              == idx[None, :]).astype(jnp.bfloat16)
    out = jax.lax.dot_general(
        cbdec_ref[...], onehot, (((0,), (0,)), ((), ())),
        preferred_element_type=jnp.float32, precision=_DEF)
    out_ref[0] = out + dec_b_ref[...]


@functools.partial(jax.jit, static_argnames=())
def kernel(x, enc_W, enc_b, codebook, dec_W, dec_b):
    b, c, d_, h, w = x.shape
    k, emb = codebook.shape
    s = d_ * h * w
    xv = x.reshape(b, c, s)
    grid = (b, s // _TILE_S)
    out = pl.pallas_call(
        _vq_kernel,
        grid=grid,
        in_specs=[
            pl.BlockSpec((1, c, _TILE_S), lambda bi, si: (bi, 0, si)),
            pl.BlockSpec((emb, c), lambda bi, si: (0, 0)),
            pl.BlockSpec((emb, 1), lambda bi, si: (0, 0)),
            pl.BlockSpec((k, emb), lambda bi, si: (0, 0)),
            pl.BlockSpec((c, emb), lambda bi, si: (0, 0)),
            pl.BlockSpec((c, 1), lambda bi, si: (0, 0)),
        ],
        out_specs=pl.BlockSpec((1, c, _TILE_S), lambda bi, si: (bi, 0, si)),
        out_shape=jax.ShapeDtypeStruct((b, c, s), jnp.float32),
        scratch_shapes=[
            pltpu.VMEM((k, emb), jnp.float32),
            pltpu.VMEM((k, 1), jnp.float32),
            pltpu.VMEM((k, c), jnp.float32),
        ],
    )(xv, enc_W, enc_b.reshape(emb, 1), codebook, dec_W, dec_b.reshape(c, 1))
    return out.reshape(b, c, d_, h, w)


# speedup vs baseline: 4.2909x; 1.0249x over previous
import functools

import jax
import jax.numpy as jnp
from jax.experimental import pallas as pl
from jax.experimental.pallas import tpu as pltpu

_TILE_S = 4096
_DEF = jax.lax.Precision.DEFAULT


def _vq_kernel(xv_ref, enc_w_ref, enc_b_ref, cb_ref, dec_w_ref, dec_b_ref,
               out_ref, cbm2_ref, cbsq_ref, cbdec_ref):
    @pl.when((pl.program_id(0) == 0) & (pl.program_id(1) == 0))
    def _init():
        cb = cb_ref[...]
        cbm2_ref[...] = cb * -2.0
        cbsq_ref[...] = jnp.sum(cb * cb, axis=1, keepdims=True)
        cbdec_ref[...] = jax.lax.dot_general(
            cb, dec_w_ref[...], (((1,), (1,)), ((), ())),
            preferred_element_type=jnp.float32, precision=_DEF)

    xv = xv_ref[0]
    zt = jax.lax.dot_general(
        enc_w_ref[...], xv, (((1,), (0,)), ((), ())),
        preferred_element_type=jnp.float32, precision=_DEF)
    zt = zt + enc_b_ref[...]
    zcm2 = jax.lax.dot_general(
        cbm2_ref[...], zt, (((1,), (0,)), ((), ())),
        preferred_element_type=jnp.float32, precision=_DEF)
    zsq = jnp.sum(zt * zt, axis=0, keepdims=True)
    dists = (zsq + zcm2) + cbsq_ref[...]
    idx = jnp.argmin(dists, axis=0)
    onehot = (jax.lax.broadcasted_iota(jnp.int32, dists.shape, 0)
              == idx[None, :]).astype(jnp.bfloat16)
    out = jax.lax.dot_general(
        cbdec_ref[...], onehot, (((0,), (0,)), ((), ())),
        preferred_element_type=jnp.float32, precision=_DEF)
    out_ref[0] = out + dec_b_ref[...]


@functools.partial(jax.jit, static_argnames=())
def kernel(x, enc_W, enc_b, codebook, dec_W, dec_b):
    b, c, d_, h, w = x.shape
    k, emb = codebook.shape
    s = d_ * h * w
    xv = x.reshape(b, c, s)
    grid = (b, s // _TILE_S)
    out = pl.pallas_call(
        _vq_kernel,
        grid=grid,
        in_specs=[
            pl.BlockSpec((1, c, _TILE_S), lambda bi, si: (bi, 0, si)),
            pl.BlockSpec((emb, c), lambda bi, si: (0, 0)),
            pl.BlockSpec((emb, 1), lambda bi, si: (0, 0)),
            pl.BlockSpec((k, emb), lambda bi, si: (0, 0)),
            pl.BlockSpec((c, emb), lambda bi, si: (0, 0)),
            pl.BlockSpec((c, 1), lambda bi, si: (0, 0)),
        ],
        out_specs=pl.BlockSpec((1, c, _TILE_S), lambda bi, si: (bi, 0, si)),
        out_shape=jax.ShapeDtypeStruct((b, c, s), jnp.float32),
        scratch_shapes=[
            pltpu.VMEM((k, emb), jnp.float32),
            pltpu.VMEM((k, 1), jnp.float32),
            pltpu.VMEM((k, c), jnp.float32),
        ],
    )(xv, enc_W, enc_b.reshape(emb, 1), codebook, dec_W, dec_b.reshape(c, 1))
    return out.reshape(b, c, d_, h, w)


# speedup vs baseline: 4.7105x; 1.0978x over previous
import functools

import jax
import jax.numpy as jnp
from jax.experimental import pallas as pl
from jax.experimental.pallas import tpu as pltpu

_TILE_S = 4096
_DEF = jax.lax.Precision.DEFAULT


def _vq_kernel(xv_ref, enc_w_ref, enc_b_ref, cb_ref, dec_w_ref, dec_b_ref,
               out_ref, cbm2_ref, cbsq_ref, cbdec_ref):
    @pl.when((pl.program_id(0) == 0) & (pl.program_id(1) == 0))
    def _init():
        cb = cb_ref[...]
        cbm2_ref[...] = cb * -2.0
        cbsq_ref[...] = jnp.sum(cb * cb, axis=1, keepdims=True)
        cbdec_ref[...] = jax.lax.dot_general(
            cb, dec_w_ref[...], (((1,), (1,)), ((), ())),
            preferred_element_type=jnp.float32, precision=_DEF)

    xv = xv_ref[0].reshape(xv_ref.shape[1], -1)
    zt = jax.lax.dot_general(
        enc_w_ref[...], xv, (((1,), (0,)), ((), ())),
        preferred_element_type=jnp.float32, precision=_DEF)
    zt = zt + enc_b_ref[...]
    zcm2 = jax.lax.dot_general(
        cbm2_ref[...], zt, (((1,), (0,)), ((), ())),
        preferred_element_type=jnp.float32, precision=_DEF)
    zsq = jnp.sum(zt * zt, axis=0, keepdims=True)
    dists = (zsq + zcm2) + cbsq_ref[...]
    idx = jnp.argmin(dists, axis=0)
    onehot = (jax.lax.broadcasted_iota(jnp.int32, dists.shape, 0)
              == idx[None, :]).astype(jnp.bfloat16)
    out = jax.lax.dot_general(
        cbdec_ref[...], onehot, (((0,), (0,)), ((), ())),
        preferred_element_type=jnp.float32, precision=_DEF)
    out_ref[0] = (out + dec_b_ref[...]).reshape(out_ref.shape[1:])


@functools.partial(jax.jit, static_argnames=())
def kernel(x, enc_W, enc_b, codebook, dec_W, dec_b):
    b, c, d_, h, w = x.shape
    k, emb = codebook.shape
    s = d_ * h * w
    dt = _TILE_S // (h * w)
    grid = (b, d_ // dt)
    out = pl.pallas_call(
        _vq_kernel,
        grid=grid,
        in_specs=[
            pl.BlockSpec((1, c, dt, h, w), lambda bi, si: (bi, 0, si, 0, 0)),
            pl.BlockSpec((emb, c), lambda bi, si: (0, 0)),
            pl.BlockSpec((emb, 1), lambda bi, si: (0, 0)),
            pl.BlockSpec((k, emb), lambda bi, si: (0, 0)),
            pl.BlockSpec((c, emb), lambda bi, si: (0, 0)),
            pl.BlockSpec((c, 1), lambda bi, si: (0, 0)),
        ],
        out_specs=pl.BlockSpec((1, c, dt, h, w), lambda bi, si: (bi, 0, si, 0, 0)),
        out_shape=jax.ShapeDtypeStruct((b, c, d_, h, w), jnp.float32),
        scratch_shapes=[
            pltpu.VMEM((k, emb), jnp.float32),
            pltpu.VMEM((k, 1), jnp.float32),
            pltpu.VMEM((k, c), jnp.float32),
        ],
    )(x, enc_W, enc_b.reshape(emb, 1), codebook, dec_W, dec_b.reshape(c, 1))
    return out


# speedup vs baseline: 4.9822x; 1.0577x over previous
import functools

import jax
import jax.numpy as jnp
from jax.experimental import pallas as pl
from jax.experimental.pallas import tpu as pltpu

_TILE_S = 4096
_DEF = jax.lax.Precision.DEFAULT


def _vq_kernel(xv_ref, enc_w_ref, enc_b_ref, cb_ref, dec_w_ref, dec_b_ref,
               out_ref, cbm2_ref, cbsq_ref, cbdec_ref):
    @pl.when((pl.program_id(0) == 0) & (pl.program_id(1) == 0))
    def _init():
        cb = cb_ref[...]
        cbm2_ref[...] = cb * -2.0
        cbsq_ref[...] = jnp.sum(cb * cb, axis=1, keepdims=True)
        cbdec_ref[...] = jax.lax.dot_general(
            cb, dec_w_ref[...], (((1,), (1,)), ((), ())),
            preferred_element_type=jnp.float32, precision=_DEF)

    xv = xv_ref[0].reshape(xv_ref.shape[1], -1)
    zt = jax.lax.dot_general(
        enc_w_ref[...], xv, (((1,), (0,)), ((), ())),
        preferred_element_type=jnp.float32, precision=_DEF)
    zt = zt + enc_b_ref[...].reshape(-1, 1)
    zcm2 = jax.lax.dot_general(
        cbm2_ref[...], zt, (((1,), (0,)), ((), ())),
        preferred_element_type=jnp.float32, precision=_DEF)
    zsq = jnp.sum(zt * zt, axis=0, keepdims=True)
    dists = (zsq + zcm2) + cbsq_ref[...]
    idx = jnp.argmin(dists, axis=0)
    onehot = (jax.lax.broadcasted_iota(jnp.int32, dists.shape, 0)
              == idx[None, :]).astype(jnp.bfloat16)
    out = jax.lax.dot_general(
        cbdec_ref[...], onehot, (((0,), (0,)), ((), ())),
        preferred_element_type=jnp.float32, precision=_DEF)
    out_ref[0] = (out + dec_b_ref[...].reshape(-1, 1)).reshape(out_ref.shape[1:])


@functools.partial(jax.jit, static_argnames=())
def kernel(x, enc_W, enc_b, codebook, dec_W, dec_b):
    b, c, d_, h, w = x.shape
    k, emb = codebook.shape
    s = d_ * h * w
    dt = _TILE_S // (h * w)
    grid = (b, d_ // dt)
    out = pl.pallas_call(
        _vq_kernel,
        grid=grid,
        in_specs=[
            pl.BlockSpec((1, c, dt, h, w), lambda bi, si: (bi, 0, si, 0, 0)),
            pl.BlockSpec((emb, c), lambda bi, si: (0, 0)),
            pl.BlockSpec((emb,), lambda bi, si: (0,)),
            pl.BlockSpec((k, emb), lambda bi, si: (0, 0)),
            pl.BlockSpec((c, emb), lambda bi, si: (0, 0)),
            pl.BlockSpec((c,), lambda bi, si: (0,)),
        ],
        out_specs=pl.BlockSpec((1, c, dt, h, w), lambda bi, si: (bi, 0, si, 0, 0)),
        out_shape=jax.ShapeDtypeStruct((b, c, d_, h, w), jnp.float32),
        scratch_shapes=[
            pltpu.VMEM((k, emb), jnp.float32),
            pltpu.VMEM((k, 1), jnp.float32),
            pltpu.VMEM((k, c), jnp.float32),
        ],
    )(x, enc_W, enc_b, codebook, dec_W, dec_b)
    return out


# speedup vs baseline: 5.1364x; 1.0310x over previous
import functools

import jax
import jax.numpy as jnp
from jax.experimental import pallas as pl
from jax.experimental.pallas import tpu as pltpu

_TILE_S = 8192
_DEF = jax.lax.Precision.DEFAULT


def _vq_kernel(xv_ref, enc_w_ref, enc_b_ref, cb_ref, dec_w_ref, dec_b_ref,
               out_ref, cbm2_ref, cbsq_ref, cbdec_ref):
    @pl.when((pl.program_id(0) == 0) & (pl.program_id(1) == 0))
    def _init():
        cb = cb_ref[...]
        cbm2_ref[...] = cb * -2.0
        cbsq_ref[...] = jnp.sum(cb * cb, axis=1, keepdims=True)
        cbdec_ref[...] = jax.lax.dot_general(
            cb, dec_w_ref[...], (((1,), (1,)), ((), ())),
            preferred_element_type=jnp.float32, precision=_DEF)

    xv = xv_ref[0].reshape(xv_ref.shape[1], -1)
    zt = jax.lax.dot_general(
        enc_w_ref[...], xv, (((1,), (0,)), ((), ())),
        preferred_element_type=jnp.float32, precision=_DEF)
    zt = zt + enc_b_ref[...].reshape(-1, 1)
    zcm2 = jax.lax.dot_general(
        cbm2_ref[...], zt, (((1,), (0,)), ((), ())),
        preferred_element_type=jnp.float32, precision=_DEF)
    zsq = jnp.sum(zt * zt, axis=0, keepdims=True)
    dists = (zsq + zcm2) + cbsq_ref[...]
    idx = jnp.argmin(dists, axis=0)
    onehot = (jax.lax.broadcasted_iota(jnp.int32, dists.shape, 0)
              == idx[None, :]).astype(jnp.bfloat16)
    out = jax.lax.dot_general(
        cbdec_ref[...], onehot, (((0,), (0,)), ((), ())),
        preferred_element_type=jnp.float32, precision=_DEF)
    out_ref[0] = (out + dec_b_ref[...].reshape(-1, 1)).reshape(out_ref.shape[1:])


@functools.partial(jax.jit, static_argnames=())
def kernel(x, enc_W, enc_b, codebook, dec_W, dec_b):
    b, c, d_, h, w = x.shape
    k, emb = codebook.shape
    s = d_ * h * w
    dt = _TILE_S // (h * w)
    grid = (b, d_ // dt)
    out = pl.pallas_call(
        _vq_kernel,
        grid=grid,
        in_specs=[
            pl.BlockSpec((1, c, dt, h, w), lambda bi, si: (bi, 0, si, 0, 0)),
            pl.BlockSpec((emb, c), lambda bi, si: (0, 0)),
            pl.BlockSpec((emb,), lambda bi, si: (0,)),
            pl.BlockSpec((k, emb), lambda bi, si: (0, 0)),
            pl.BlockSpec((c, emb), lambda bi, si: (0, 0)),
            pl.BlockSpec((c,), lambda bi, si: (0,)),
        ],
        out_specs=pl.BlockSpec((1, c, dt, h, w), lambda bi, si: (bi, 0, si, 0, 0)),
        out_shape=jax.ShapeDtypeStruct((b, c, d_, h, w), jnp.float32),
        scratch_shapes=[
            pltpu.VMEM((k, emb), jnp.float32),
            pltpu.VMEM((k, 1), jnp.float32),
            pltpu.VMEM((k, c), jnp.float32),
        ],
    )(x, enc_W, enc_b, codebook, dec_W, dec_b)
    return out
